# baseline (speedup 1.0000x reference)
import functools

import jax
import jax.numpy as jnp
from jax.experimental import pallas as pl
from jax.experimental.pallas import tpu as pltpu

H = 1024
I = 4096
E = 8
R = 8
ER = E * R
SCALE = 16.0 / 8.0
TN = 512
CK = 2048
NCK = I // CK
RCH = 128


def _expand_gates(logits):
    m = jnp.max(logits, axis=-1, keepdims=True)
    ex = jnp.exp(logits - m)
    gates = ex / jnp.sum(ex, axis=-1, keepdims=True)
    eidx = jax.lax.broadcasted_iota(jnp.int32, (E, ER), 0)
    cidx = jax.lax.broadcasted_iota(jnp.int32, (E, ER), 1)
    sel = jnp.where(cidx // R == eidx, SCALE, 0.0).astype(jnp.float32)
    return jnp.dot(gates, sel, preferred_element_type=jnp.float32)


def _stage_cast(src_hbm, dst_vmem, st, sem, nrows, rch):
    nch = nrows // rch

    def cp(c, slot):
        return pltpu.make_async_copy(
            src_hbm.at[pl.ds(c * rch, rch), :], st.at[slot], sem.at[slot])

    cp(0, 0).start()

    def body(c, _):
        slot = jax.lax.rem(c, 2)
        nslot = jax.lax.rem(c + 1, 2)

        @pl.when(c + 1 < nch)
        def _():
            cp(c + 1, nslot).start()

        cp(c, slot).wait()
        dst_vmem[pl.ds(c * rch, rch), :] = st[slot].astype(jnp.bfloat16)
        return 0

    jax.lax.fori_loop(0, nch, body, 0)


def _fused_kernel(x_ref, wg_hbm, wu_hbm, wd_hbm, guc_ref, dc_ref,
                  gbc_ref, ubc_ref, dbc_ref, out_ref,
                  wg_ref, wu_ref, wd_ref, st_ref, std_ref, sem):
    @pl.when(pl.program_id(0) == 0)
    def _prologue():
        _stage_cast(wg_hbm, wg_ref, st_ref, sem, H, RCH)
        _stage_cast(wu_hbm, wu_ref, st_ref, sem, H, RCH)
        _stage_cast(wd_hbm, wd_ref, std_ref, sem, I, 2 * RCH)

    xb = x_ref[...].astype(jnp.bfloat16)

    r = jnp.dot(xb, guc_ref[...], preferred_element_type=jnp.float32)
    sl0 = pl.ds(0, CK)
    g_pend = jnp.dot(xb, wg_ref[:, sl0], preferred_element_type=jnp.float32)
    u_pend = jnp.dot(xb, wu_ref[:, sl0], preferred_element_type=jnp.float32)
    whg = (r[:, 16:80] * _expand_gates(r[:, 0:8])).astype(jnp.bfloat16)
    whu = (r[:, 80:144] * _expand_gates(r[:, 8:16])).astype(jnp.bfloat16)

    out_acc = jnp.zeros((TN, H), jnp.float32)
    dr_acc = jnp.zeros((TN, E + ER), jnp.float32)
    for k in range(NCK):
        sl = pl.ds(k * CK, CK)
        g_k = g_pend + jnp.dot(whg, gbc_ref[:, sl],
                               preferred_element_type=jnp.float32)
        u_k = u_pend + jnp.dot(whu, ubc_ref[:, sl],
                               preferred_element_type=jnp.float32)
        if k + 1 < NCK:
            sln = pl.ds((k + 1) * CK, CK)
            g_pend = jnp.dot(xb, wg_ref[:, sln],
                             preferred_element_type=jnp.float32)
            u_pend = jnp.dot(xb, wu_ref[:, sln],
                             preferred_element_type=jnp.float32)
        inter_k = (g_k * jax.lax.logistic(g_k) * u_k).astype(jnp.bfloat16)
        out_acc += jnp.dot(inter_k, wd_ref[sl, :],
                           preferred_element_type=jnp.float32)
        dr_acc += jnp.dot(inter_k, dc_ref[sl, :],
                          preferred_element_type=jnp.float32)

    whd = (dr_acc[:, E:] * _expand_gates(dr_acc[:, :E])).astype(jnp.bfloat16)
    out_ref[...] = out_acc + jnp.dot(whd, dbc_ref[...],
                                     preferred_element_type=jnp.float32)


@functools.partial(jax.jit, static_argnames=())
def kernel(x, Wg, Wu, Wd, g_Wr, g_A, g_B, u_Wr, u_A, u_B, d_Wr, d_A, d_B):
    Bb, Ss, Hh = x.shape
    N = Bb * Ss
    xf = x.reshape(N, Hh)

    def acat(A):
        return A.transpose(1, 0, 2).reshape(A.shape[1], ER)

    guc = jnp.concatenate([g_Wr, u_Wr, acat(g_A), acat(u_A)],
                          axis=1).astype(jnp.bfloat16)
    dc = jnp.concatenate([d_Wr, acat(d_A)], axis=1).astype(jnp.bfloat16)
    gbc = g_B.reshape(ER, I).astype(jnp.bfloat16)
    ubc = u_B.reshape(ER, I).astype(jnp.bfloat16)
    dbc = d_B.reshape(ER, H).astype(jnp.bfloat16)

    full = lambda shape: pl.BlockSpec(shape, lambda i: (0, 0))
    hbm = pl.BlockSpec(memory_space=pl.ANY)
    out = pl.pallas_call(
        _fused_kernel,
        grid=(N // TN,),
        in_specs=[
            pl.BlockSpec((TN, H), lambda i: (i, 0)),
            hbm, hbm, hbm,
            full((H, 2 * E + 2 * ER)), full((I, E + ER)),
            full((ER, I)), full((ER, I)), full((ER, H)),
        ],
        out_specs=pl.BlockSpec((TN, H), lambda i: (i, 0)),
        out_shape=jax.ShapeDtypeStruct((N, H), jnp.float32),
        scratch_shapes=[
            pltpu.VMEM((H, I), jnp.bfloat16),
            pltpu.VMEM((H, I), jnp.bfloat16),
            pltpu.VMEM((I, H), jnp.bfloat16),
            pltpu.VMEM((2, RCH, I), jnp.float32),
            pltpu.VMEM((2, 2 * RCH, H), jnp.float32),
            pltpu.SemaphoreType.DMA((2,)),
        ],
    )(xf, Wg, Wu, Wd, guc, dc, gbc, ubc, dbc)
    return out.reshape(Bb, Ss, Hh)


# speedup vs baseline: 1.0008x; 1.0008x over previous
import functools

import jax
import jax.numpy as jnp
from jax.experimental import pallas as pl
from jax.experimental.pallas import tpu as pltpu

H = 1024
I = 4096
E = 8
R = 8
ER = E * R
SCALE = 16.0 / 8.0
TN = 512
CK = 2048
NCK = I // CK
RCH = 128


def _expand_gates(logits):
    m = jnp.max(logits, axis=-1, keepdims=True)
    ex = jnp.exp(logits - m)
    gates = ex / jnp.sum(ex, axis=-1, keepdims=True)
    eidx = jax.lax.broadcasted_iota(jnp.int32, (E, ER), 0)
    cidx = jax.lax.broadcasted_iota(jnp.int32, (E, ER), 1)
    sel = jnp.where(cidx // R == eidx, SCALE, 0.0).astype(jnp.float32)
    return jnp.dot(gates, sel, preferred_element_type=jnp.float32)


def _stage_cast(src_hbm, dst_vmem, st, sem, nrows, rch):
    nch = nrows // rch

    def cp(c, slot):
        return pltpu.make_async_copy(
            src_hbm.at[pl.ds(c * rch, rch), :], st.at[slot], sem.at[slot])

    cp(0, 0).start()

    def body(c, _):
        slot = jax.lax.rem(c, 2)
        nslot = jax.lax.rem(c + 1, 2)

        @pl.when(c + 1 < nch)
        def _():
            cp(c + 1, nslot).start()

        cp(c, slot).wait()
        dst_vmem[pl.ds(c * rch, rch), :] = st[slot].astype(jnp.bfloat16)
        return 0

    jax.lax.fori_loop(0, nch, body, 0)


def _fused_kernel(x_ref, wg_hbm, wu_hbm, wd_hbm, guc_ref, dc_ref,
                  gbc_ref, ubc_ref, dbc_ref, out_ref,
                  wg_ref, wu_ref, wd_ref, st_ref, std_ref, sem):
    @pl.when(pl.program_id(0) == 0)
    def _prologue():
        _stage_cast(wg_hbm, wg_ref, st_ref, sem, H, RCH)
        _stage_cast(wu_hbm, wu_ref, st_ref, sem, H, RCH)
        _stage_cast(wd_hbm, wd_ref, std_ref, sem, I, 2 * RCH)

    xb = x_ref[...].astype(jnp.bfloat16)

    r = jnp.dot(xb, guc_ref[...], preferred_element_type=jnp.float32)
    sl0 = pl.ds(0, CK)
    g_pend = jnp.dot(xb, wg_ref[:, sl0], preferred_element_type=jnp.float32)
    u_pend = jnp.dot(xb, wu_ref[:, sl0], preferred_element_type=jnp.float32)
    whg = (r[:, 16:80] * _expand_gates(r[:, 0:8])).astype(jnp.bfloat16)
    whu = (r[:, 80:144] * _expand_gates(r[:, 8:16])).astype(jnp.bfloat16)

    out_acc = jnp.zeros((TN, H), jnp.float32)
    dr_acc = jnp.zeros((TN, E + ER), jnp.float32)
    for k in range(NCK):
        sl = pl.ds(k * CK, CK)
        g_k = g_pend + jnp.dot(whg, gbc_ref[:, sl],
                               preferred_element_type=jnp.float32)
        u_k = u_pend + jnp.dot(whu, ubc_ref[:, sl],
                               preferred_element_type=jnp.float32)
        if k + 1 < NCK:
            sln = pl.ds((k + 1) * CK, CK)
            g_pend = jnp.dot(xb, wg_ref[:, sln],
                             preferred_element_type=jnp.float32)
            u_pend = jnp.dot(xb, wu_ref[:, sln],
                             preferred_element_type=jnp.float32)
        gb_k = g_k.astype(jnp.bfloat16)
        inter_k = gb_k * jax.lax.logistic(gb_k) * u_k.astype(jnp.bfloat16)
        out_acc += jnp.dot(inter_k, wd_ref[sl, :],
                           preferred_element_type=jnp.float32)
        dr_acc += jnp.dot(inter_k, dc_ref[sl, :],
                          preferred_element_type=jnp.float32)

    whd = (dr_acc[:, E:] * _expand_gates(dr_acc[:, :E])).astype(jnp.bfloat16)
    out_ref[...] = out_acc + jnp.dot(whd, dbc_ref[...],
                                     preferred_element_type=jnp.float32)


@functools.partial(jax.jit, static_argnames=())
def kernel(x, Wg, Wu, Wd, g_Wr, g_A, g_B, u_Wr, u_A, u_B, d_Wr, d_A, d_B):
    Bb, Ss, Hh = x.shape
    N = Bb * Ss
    xf = x.reshape(N, Hh)

    def acat(A):
        return A.transpose(1, 0, 2).reshape(A.shape[1], ER)

    guc = jnp.concatenate([g_Wr, u_Wr, acat(g_A), acat(u_A)],
                          axis=1).astype(jnp.bfloat16)
    dc = jnp.concatenate([d_Wr, acat(d_A)], axis=1).astype(jnp.bfloat16)
    gbc = g_B.reshape(ER, I).astype(jnp.bfloat16)
    ubc = u_B.reshape(ER, I).astype(jnp.bfloat16)
    dbc = d_B.reshape(ER, H).astype(jnp.bfloat16)

    full = lambda shape: pl.BlockSpec(shape, lambda i: (0, 0))
    hbm = pl.BlockSpec(memory_space=pl.ANY)
    out = pl.pallas_call(
        _fused_kernel,
        grid=(N // TN,),
        in_specs=[
            pl.BlockSpec((TN, H), lambda i: (i, 0)),
            hbm, hbm, hbm,
            full((H, 2 * E + 2 * ER)), full((I, E + ER)),
            full((ER, I)), full((ER, I)), full((ER, H)),
        ],
        out_specs=pl.BlockSpec((TN, H), lambda i: (i, 0)),
        out_shape=jax.ShapeDtypeStruct((N, H), jnp.float32),
        scratch_shapes=[
            pltpu.VMEM((H, I), jnp.bfloat16),
            pltpu.VMEM((H, I), jnp.bfloat16),
            pltpu.VMEM((I, H), jnp.bfloat16),
            pltpu.VMEM((2, RCH, I), jnp.float32),
            pltpu.VMEM((2, 2 * RCH, H), jnp.float32),
            pltpu.SemaphoreType.DMA((2,)),
        ],
    )(xf, Wg, Wu, Wd, guc, dc, gbc, ubc, dbc)
    return out.reshape(Bb, Ss, Hh)
